# Initial kernel scaffold; baseline (speedup 1.0000x reference)
import functools

import jax
import jax.numpy as jnp
from jax import lax
from jax.experimental import pallas as pl
from jax.experimental.pallas import tpu as pltpu
from jax.experimental.pallas import tpu_sc as plsc

_F = 4
_V1 = 100001
_D = 32
_B = 4096
_L = 50
_NW = 32
_BPW = _B // _NW
_G = 64
_NR = _BPW // _G

_mesh = plsc.VectorSubcoreMesh(core_axis_name="c", subcore_axis_name="s")


@functools.partial(
    pl.kernel,
    out_type=jax.ShapeDtypeStruct((_B * _F * _D,), jnp.float32),
    mesh=_mesh,
    scratch_types=[
        pltpu.VMEM((_G * _L,), jnp.int32),
        pltpu.VMEM((_G * _L, _D), jnp.float32),
        pltpu.VMEM((_G * _F * _D,), jnp.float32),
        pltpu.SemaphoreType.DMA,
    ],
)
def _tsf_pool(w_hbm, x_hbm, out_hbm, idx_v, rows_v, out_v, sem):
    wid = lax.axis_index("s") * 2 + lax.axis_index("c")
    for c in range(_NR):
        b0 = wid * _BPW + c * _G
        for f in range(_F):
            src = (f * _B) * _L + b0 * _L
            pltpu.sync_copy(x_hbm.at[pl.ds(src, _G * _L)], idx_v)

            off = f * _V1
            if off:
                def _add_off(i, _):
                    v = idx_v[pl.ds(i * 16, 16)]
                    idx_v[pl.ds(i * 16, 16)] = v + off
                    return 0
                lax.fori_loop(0, _G * _L // 16, _add_off, 0)

            pltpu.async_copy(w_hbm.at[idx_v], rows_v, sem).wait()

            def _pool_one(g, _):
                r0 = g * _L
                acc0 = rows_v[r0, pl.ds(0, 16)]
                acc1 = rows_v[r0, pl.ds(16, 16)]
                for j in range(1, _L):
                    acc0 = acc0 + rows_v[r0 + j, pl.ds(0, 16)]
                    acc1 = acc1 + rows_v[r0 + j, pl.ds(16, 16)]
                ones = jnp.zeros((16,), jnp.float32)
                for k in range(3):
                    v = idx_v[pl.ds(r0 + k * 16, 16)]
                    ones = ones + jnp.where(v != off, 1.0, 0.0).astype(jnp.float32)
                cnt = jnp.sum(ones)
                s0 = idx_v[r0 + 48]
                s1 = idx_v[r0 + 49]
                cnt = (cnt
                       + jnp.where(s0 != off, 1.0, 0.0)
                       + jnp.where(s1 != off, 1.0, 0.0))
                rinv = 1.0 / cnt
                o = (g * _F + f) * _D
                out_v[pl.ds(o, 16)] = acc0 * rinv
                out_v[pl.ds(o + 16, 16)] = acc1 * rinv
                return 0
            lax.fori_loop(0, _G, _pool_one, 0)

        pltpu.sync_copy(out_v, out_hbm.at[pl.ds(b0 * _F * _D, _G * _F * _D)])


def kernel(x, W):
    x1 = x.reshape(-1)
    w1 = W.reshape(_F * _V1, _D)
    out = _tsf_pool(w1, x1)
    return out.reshape(_B, _F, _D)


# speedup vs baseline: 8.6121x; 8.6121x over previous
import functools

import jax
import jax.numpy as jnp
from jax import lax
from jax.experimental import pallas as pl
from jax.experimental.pallas import tpu as pltpu
from jax.experimental.pallas import tpu_sc as plsc

_F = 4
_V1 = 100001
_D = 32
_B = 4096
_L = 50
_NW = 32
_BPW = _B // _NW
_G = 64
_NR = _BPW // _G

_mesh = plsc.VectorSubcoreMesh(core_axis_name="c", subcore_axis_name="s")


@functools.partial(
    pl.kernel,
    out_type=jax.ShapeDtypeStruct((_B * _F * _D,), jnp.float32),
    mesh=_mesh,
    scratch_types=[
        pltpu.VMEM((_G * _L,), jnp.int32),
        pltpu.VMEM((_G * _L, _D), jnp.float32),
        pltpu.VMEM((_G * _F * _D,), jnp.float32),
        pltpu.SemaphoreType.DMA,
    ],
    compiler_params=pltpu.CompilerParams(use_tc_tiling_on_sc=False),
)
def _tsf_pool(w_hbm, x_hbm, out_hbm, idx_v, rows_v, out_v, sem):
    wid = lax.axis_index("s") * 2 + lax.axis_index("c")
    for c in range(_NR):
        b0 = wid * _BPW + c * _G
        for f in range(_F):
            src = (f * _B) * _L + b0 * _L
            pltpu.sync_copy(x_hbm.at[pl.ds(src, _G * _L)], idx_v)

            off = f * _V1
            if off:
                def _add_off(i, _):
                    v = idx_v[pl.ds(i * 16, 16)]
                    idx_v[pl.ds(i * 16, 16)] = v + off
                    return 0
                lax.fori_loop(0, _G * _L // 16, _add_off, 0)

            pltpu.async_copy(w_hbm.at[idx_v], rows_v, sem).wait()

            def _pool_one(g, _):
                r0 = g * _L
                acc0 = rows_v[r0, pl.ds(0, 16)]
                acc1 = rows_v[r0, pl.ds(16, 16)]
                for j in range(1, _L):
                    acc0 = acc0 + rows_v[r0 + j, pl.ds(0, 16)]
                    acc1 = acc1 + rows_v[r0 + j, pl.ds(16, 16)]
                ones = jnp.zeros((16,), jnp.float32)
                for k in range(3):
                    v = idx_v[pl.ds(r0 + k * 16, 16)]
                    ones = ones + jnp.where(v != off, 1.0, 0.0).astype(jnp.float32)
                lane = lax.iota(jnp.int32, 16)
                v3 = idx_v[pl.ds(r0 + 34, 16)]
                ones = ones + jnp.where((lane >= 14) & (v3 != off), 1.0, 0.0)
                for s in (8, 4, 2, 1):
                    perm = jnp.bitwise_xor(lane, s)
                    ones = ones + ones.at[perm].get(mode="promise_in_bounds")
                rinv = 1.0 / ones
                o = (g * _F + f) * _D
                out_v[pl.ds(o, 16)] = acc0 * rinv
                out_v[pl.ds(o + 16, 16)] = acc1 * rinv
                return 0
            lax.fori_loop(0, _G, _pool_one, 0)

        pltpu.sync_copy(out_v, out_hbm.at[pl.ds(b0 * _F * _D, _G * _F * _D)])


def kernel(x, W):
    x1 = x.reshape(-1)
    w1 = W.reshape(_F * _V1, _D)
    out = _tsf_pool(w1, x1)
    return out.reshape(_B, _F, _D)


# speedup vs baseline: 9.2591x; 1.0751x over previous
import functools

import jax
import jax.numpy as jnp
from jax import lax
from jax.experimental import pallas as pl
from jax.experimental.pallas import tpu as pltpu
from jax.experimental.pallas import tpu_sc as plsc

_F = 4
_V1 = 100001
_D = 32
_B = 4096
_L = 50
_NW = 32
_BPW = _B // _NW
_G = 32
_NC = _BPW // _G
_NROUND = _NC * _F

_mesh = plsc.VectorSubcoreMesh(core_axis_name="c", subcore_axis_name="s")


@functools.partial(
    pl.kernel,
    out_type=jax.ShapeDtypeStruct((_B * _F * _D,), jnp.float32),
    mesh=_mesh,
    scratch_types=[
        pltpu.VMEM((_G * _L,), jnp.int32),
        pltpu.VMEM((_G * _L,), jnp.int32),
        pltpu.VMEM((_G * _L, _D), jnp.float32),
        pltpu.VMEM((_G * _L, _D), jnp.float32),
        pltpu.VMEM((_G * _F * _D,), jnp.float32),
        pltpu.SemaphoreType.DMA,
        pltpu.SemaphoreType.DMA,
    ],
    compiler_params=pltpu.CompilerParams(use_tc_tiling_on_sc=False),
)
def _tsf_pool(w_hbm, x_hbm, out_hbm, idx0, idx1, rows0, rows1, out_v,
              sem0, sem1):
    wid = lax.axis_index("s") * 2 + lax.axis_index("c")
    idx = (idx0, idx1)
    rows = (rows0, rows1)
    sems = (sem0, sem1)

    def stage(i):
        s = i % 2
        c, f = divmod(i, _F)
        b0 = wid * _BPW + c * _G
        src = (f * _B + b0) * _L
        pltpu.sync_copy(x_hbm.at[pl.ds(src, _G * _L)], idx[s])
        off = f * _V1
        if off:
            def _add_off(k, _):
                v = idx[s][pl.ds(k * 16, 16)]
                idx[s][pl.ds(k * 16, 16)] = v + off
                return 0
            lax.fori_loop(0, _G * _L // 16, _add_off, 0)
        return pltpu.async_copy(w_hbm.at[idx[s]], rows[s], sems[s])

    pending = stage(0)
    for i in range(_NROUND):
        s = i % 2
        c, f = divmod(i, _F)
        desc = pending
        if i + 1 < _NROUND:
            pending = stage(i + 1)
        desc.wait()

        off = f * _V1
        idx_s = idx[s]
        rows_s = rows[s]

        def _pool_one(g, _):
            r0 = g * _L
            acc0 = rows_s[r0, pl.ds(0, 16)]
            acc1 = rows_s[r0, pl.ds(16, 16)]
            for j in range(1, _L):
                acc0 = acc0 + rows_s[r0 + j, pl.ds(0, 16)]
                acc1 = acc1 + rows_s[r0 + j, pl.ds(16, 16)]
            ones = jnp.zeros((16,), jnp.float32)
            for k in range(3):
                v = idx_s[pl.ds(r0 + k * 16, 16)]
                ones = ones + jnp.where(v != off, 1.0, 0.0).astype(jnp.float32)
            lane = lax.iota(jnp.int32, 16)
            v3 = idx_s[pl.ds(r0 + 34, 16)]
            ones = ones + jnp.where((lane >= 14) & (v3 != off), 1.0, 0.0)
            for st in (8, 4, 2, 1):
                perm = jnp.bitwise_xor(lane, st)
                ones = ones + ones.at[perm].get(mode="promise_in_bounds")
            rinv = 1.0 / ones
            o = (g * _F + f) * _D
            out_v[pl.ds(o, 16)] = acc0 * rinv
            out_v[pl.ds(o + 16, 16)] = acc1 * rinv
            return 0
        lax.fori_loop(0, _G, _pool_one, 0)

        if f == _F - 1:
            b0 = wid * _BPW + c * _G
            pltpu.sync_copy(out_v,
                            out_hbm.at[pl.ds(b0 * _F * _D, _G * _F * _D)])


def kernel(x, W):
    x1 = x.reshape(-1)
    w1 = W.reshape(_F * _V1, _D)
    out = _tsf_pool(w1, x1)
    return out.reshape(_B, _F, _D)
